# Initial kernel scaffold; baseline (speedup 1.0000x reference)
import jax
import jax.numpy as jnp
from jax.experimental import pallas as pl


def kernel(x, adj_indices, adj_values, W1, b1, W2, b2):
    raise NotImplementedError("write your pallas kernel here")


# speedup vs baseline: 2.4336x; 2.4336x over previous
import functools

import jax
import jax.numpy as jnp
from jax import lax
from jax.experimental import pallas as pl
from jax.experimental.pallas import tpu as pltpu
import jax.experimental.pallas.tpu_sc as plsc

N = 65536
NFEAT = 64
NHID = 64
NCLASS = 40
NCPAD = 48

NNZ = 4294967
B = 128
TILES = 16
E = 2098 * TILES * B
NBATCH = E // (TILES * B)
RPT = N // TILES

_MM_BLK = 512


def _mm1_body(x_ref, w_ref, o_ref):
    res = jnp.dot(x_ref[...], w_ref[...], preferred_element_type=jnp.float32)
    for j in range(4):
        o_ref[j] = res[:, j * 16:(j + 1) * 16]


def _mm2_body(g_ref, w_ref, b_ref, o_ref):
    h = jnp.concatenate([g_ref[j] for j in range(4)], axis=1)
    bias = jnp.dot(b_ref[...], w_ref[...], preferred_element_type=jnp.float32)
    res = jnp.dot(h, w_ref[...], preferred_element_type=jnp.float32)
    res = res + bias
    for j in range(3):
        o_ref[j] = res[:, j * 16:(j + 1) * 16]


def _asm_body(s_ref, o_ref):
    res = jnp.concatenate([s_ref[j] for j in range(3)], axis=1)
    o_ref[...] = res[:, :NCLASS]


def _asm(s):
    return pl.pallas_call(
        _asm_body,
        grid=(N // _MM_BLK,),
        in_specs=[pl.BlockSpec((3, _MM_BLK, 16), lambda i: (0, i, 0))],
        out_specs=pl.BlockSpec((_MM_BLK, NCLASS), lambda i: (i, 0)),
        out_shape=jax.ShapeDtypeStruct((N, NCLASS), jnp.float32),
    )(s)


def _mm1(x, w1):
    return pl.pallas_call(
        _mm1_body,
        grid=(N // _MM_BLK,),
        in_specs=[
            pl.BlockSpec((_MM_BLK, NFEAT), lambda i: (i, 0)),
            pl.BlockSpec((NFEAT, NHID), lambda i: (0, 0)),
        ],
        out_specs=pl.BlockSpec((4, _MM_BLK, 16), lambda i: (0, i, 0)),
        out_shape=jax.ShapeDtypeStruct((4, N, 16), jnp.float32),
    )(x, w1)


def _mm2(g, w2p, b1):
    return pl.pallas_call(
        _mm2_body,
        grid=(N // _MM_BLK,),
        in_specs=[
            pl.BlockSpec((4, _MM_BLK, 16), lambda i: (0, i, 0)),
            pl.BlockSpec((NHID, NCPAD), lambda i: (0, 0)),
            pl.BlockSpec((1, NHID), lambda i: (0, 0)),
        ],
        out_specs=pl.BlockSpec((3, _MM_BLK, 16), lambda i: (0, i, 0)),
        out_shape=jax.ShapeDtypeStruct((3, N, 16), jnp.float32),
    )(g, w2p, b1)


def _spmm_body(nslab, table, rows, cols, vals, init, out,
               acc, col_idx, row_idx, val_b, gath, sem):
    cid = lax.axis_index("c")
    sid = lax.axis_index("s")
    ebase = sid * (E // TILES)

    for jj in range(2):
        j = cid * 2 + jj

        @pl.when(j < nslab)
        def _pass():
            pltpu.sync_copy(init.at[j], acc.at[pl.ds(sid * RPT, RPT)])
            plsc.subcore_barrier()

            def batch(b, _):
                off = ebase + b * B
                pltpu.sync_copy(cols.at[pl.ds(off, B)], col_idx)
                pltpu.sync_copy(rows.at[pl.ds(off, B)], row_idx)
                pltpu.sync_copy(vals.at[pl.ds(off, B)], val_b)
                for c in range(B // 16):
                    sl = pl.ds(c * 16, 16)
                    col_idx[sl] = col_idx[sl] + j * N
                pltpu.async_copy(table.at[col_idx], gath, sem).wait()

                def scale(cc, _):
                    for ii in range(16):
                        i = cc * 16 + ii
                        sv = plsc.load_gather(
                            val_b, [jnp.full((16,), i, jnp.int32)])
                        gath[i, :] = gath[i, :] * sv
                    return 0
                lax.fori_loop(0, B // 16, scale, 0)

                pltpu.sync_copy(gath, acc.at[row_idx], add=True)
                return 0

            lax.fori_loop(0, NBATCH, batch, 0)
            plsc.subcore_barrier()
            pltpu.sync_copy(acc.at[pl.ds(sid * RPT, RPT)],
                            out.at[j, pl.ds(sid * RPT, RPT)])


def _spmm(nslab, table_flat, rows, cols, vals, init):
    mesh = plsc.VectorSubcoreMesh(core_axis_name="c", subcore_axis_name="s",
                                  num_cores=2, num_subcores=TILES)
    fn = pl.kernel(
        functools.partial(_spmm_body, nslab),
        out_type=jax.ShapeDtypeStruct((nslab, N, 16), jnp.float32),
        mesh=mesh,
        compiler_params=pltpu.CompilerParams(needs_layout_passes=False,
                                             use_tc_tiling_on_sc=False),
        scratch_types=[
            pltpu.VMEM_SHARED((N, 16), jnp.float32),
            pltpu.VMEM((B,), jnp.int32),
            pltpu.VMEM((B,), jnp.int32),
            pltpu.VMEM((B,), jnp.float32),
            pltpu.VMEM((B, 16), jnp.float32),
            pltpu.SemaphoreType.DMA,
        ],
    )
    return fn(table_flat, rows, cols, vals, init)


def kernel(x, adj_indices, adj_values, W1, b1, W2, b2):
    rows = adj_indices[0]
    cols = adj_indices[1]
    pad = E - NNZ
    rows_p = jnp.pad(rows, (0, pad))
    cols_p = jnp.pad(cols, (0, pad))
    vals_p = jnp.pad(adj_values, (0, pad))

    w2p = jnp.pad(W2, ((0, 0), (0, NCPAD - NCLASS)))
    b2p = jnp.pad(b2, (0, NCPAD - NCLASS))

    xw = _mm1(x, W1)
    init0 = jnp.zeros((4, RPT, 16), jnp.float32)
    g1 = _spmm(4, xw.reshape(4 * N, 16), rows_p, cols_p, vals_p, init0)

    t = _mm2(g1, w2p, b1.reshape(1, NHID))
    init2 = jnp.broadcast_to(b2p.reshape(3, 1, 16), (3, RPT, 16))
    out3 = _spmm(3, t.reshape(3 * N, 16), rows_p, cols_p, vals_p, init2)

    return _asm(out3)


# speedup vs baseline: 9.2899x; 3.8173x over previous
import functools

import jax
import jax.numpy as jnp
from jax import lax
from jax.experimental import pallas as pl
from jax.experimental.pallas import tpu as pltpu
import jax.experimental.pallas.tpu_sc as plsc

N = 65536
NFEAT = 64
NHID = 64
NCLASS = 40
NCPAD = 48

NNZ = 4294967
B = 128
K = 8
CHUNK = K * B
TILES = 16
NCHUNK = 264
E = NCHUNK * TILES * CHUNK
ER = E // B
TROW = E // TILES // B
RPT = N // TILES

_MM_BLK = 512


def _mm1_body(x_ref, w_ref, o_ref):
    res = jnp.dot(x_ref[...], w_ref[...], preferred_element_type=jnp.float32)
    for j in range(4):
        o_ref[j] = res[:, j * 16:(j + 1) * 16]


def _mm2_body(g_ref, w_ref, b_ref, o_ref):
    h = jnp.concatenate([g_ref[j] for j in range(4)], axis=1)
    bias = jnp.dot(b_ref[...], w_ref[...], preferred_element_type=jnp.float32)
    res = jnp.dot(h, w_ref[...], preferred_element_type=jnp.float32)
    res = res + bias
    for j in range(3):
        o_ref[j] = res[:, j * 16:(j + 1) * 16]


def _asm_body(s_ref, o_ref):
    res = jnp.concatenate([s_ref[j] for j in range(3)], axis=1)
    o_ref[...] = res[:, :NCLASS]


def _asm(s):
    return pl.pallas_call(
        _asm_body,
        grid=(N // _MM_BLK,),
        in_specs=[pl.BlockSpec((3, _MM_BLK, 16), lambda i: (0, i, 0))],
        out_specs=pl.BlockSpec((_MM_BLK, NCLASS), lambda i: (i, 0)),
        out_shape=jax.ShapeDtypeStruct((N, NCLASS), jnp.float32),
    )(s)


def _mm1(x, w1):
    return pl.pallas_call(
        _mm1_body,
        grid=(N // _MM_BLK,),
        in_specs=[
            pl.BlockSpec((_MM_BLK, NFEAT), lambda i: (i, 0)),
            pl.BlockSpec((NFEAT, NHID), lambda i: (0, 0)),
        ],
        out_specs=pl.BlockSpec((4, _MM_BLK, 16), lambda i: (0, i, 0)),
        out_shape=jax.ShapeDtypeStruct((4, N, 16), jnp.float32),
    )(x, w1)


def _mm2(g, w2p, b1):
    return pl.pallas_call(
        _mm2_body,
        grid=(N // _MM_BLK,),
        in_specs=[
            pl.BlockSpec((4, _MM_BLK, 16), lambda i: (0, i, 0)),
            pl.BlockSpec((NHID, NCPAD), lambda i: (0, 0)),
            pl.BlockSpec((1, NHID), lambda i: (0, 0)),
        ],
        out_specs=pl.BlockSpec((3, _MM_BLK, 16), lambda i: (0, i, 0)),
        out_shape=jax.ShapeDtypeStruct((3, N, 16), jnp.float32),
    )(g, w2p, b1)


def _spmm_body(nslab, table, rows, cols, vals, init, out,
               acc, rbuf, cbuf, vbuf, gath, semg0, semg1, semi0, semi1):
    cid = lax.axis_index("c")
    sid = lax.axis_index("s")
    rbase = sid * TROW
    semg = (semg0, semg1)
    semi = (semi0, semi1)

    def idx_copies(c, p, sem):
        src = pl.ds(rbase + c * K, K)
        return (
            pltpu.async_copy(rows.at[src], rbuf.at[p], sem),
            pltpu.async_copy(cols.at[src], cbuf.at[p], sem),
            pltpu.async_copy(vals.at[src], vbuf.at[p], sem),
        )

    def drain_idx(p, sem):
        src = pl.ds(rbase, K)
        pltpu.make_async_copy(rows.at[src], rbuf.at[p], sem).wait()
        pltpu.make_async_copy(cols.at[src], cbuf.at[p], sem).wait()
        pltpu.make_async_copy(vals.at[src], vbuf.at[p], sem).wait()

    def add_slab_offset(p, j):
        def go(q, _):
            k = q // 8
            sl = pl.ds((q % 8) * 16, 16)
            cbuf[p, k, sl] = cbuf[p, k, sl] + j * N
            return 0
        lax.fori_loop(0, 64, go, 0)

    def fire_gathers(p, sem):
        for k in range(K):
            pltpu.async_copy(table.at[cbuf.at[p, k]],
                             gath.at[p, pl.ds(k * B, B)], sem)

    def drain_gathers(p, sem):
        for k in range(K):
            pltpu.make_async_copy(table.at[cbuf.at[p, k]],
                                  gath.at[p, pl.ds(k * B, B)], sem).wait()

    def mult_scatter(p):
        def per_sub(k, _):
            def per16(c16, _):
                v16 = vbuf[p, k, pl.ds(c16 * 16, 16)]
                for ii in range(16):
                    i = k * B + c16 * 16 + ii
                    bc = v16[jnp.full((16,), ii, jnp.int32)]
                    gath[p, i, :] = gath[p, i, :] * bc
                return 0
            lax.fori_loop(0, B // 16, per16, 0)
            pltpu.sync_copy(gath.at[p, pl.ds(k * B, B)],
                            acc.at[rbuf.at[p, k]], add=True)
            return 0
        lax.fori_loop(0, K, per_sub, 0)

    def do_chunk(c, cur, nxt, j):
        drain_idx(nxt, semi[nxt])
        add_slab_offset(nxt, j)
        fire_gathers(nxt, semg[nxt])
        drain_gathers(cur, semg[cur])
        mult_scatter(cur)
        cc = jnp.minimum(c + 2, NCHUNK - 1)
        idx_copies(cc, cur, semi[cur])

    def one_pass(jj, _):
        j = cid * 2 + jj

        @pl.when(j < nslab)
        def _pass():
            pltpu.sync_copy(init.at[j], acc.at[pl.ds(sid * RPT, RPT)])
            plsc.subcore_barrier()

            idx_copies(0, 0, semi[0])
            drain_idx(0, semi[0])
            add_slab_offset(0, j)
            fire_gathers(0, semg[0])
            idx_copies(1, 1, semi[1])

            def pair(t, _):
                do_chunk(2 * t, 0, 1, j)
                do_chunk(2 * t + 1, 1, 0, j)
                return 0
            lax.fori_loop(0, NCHUNK // 2, pair, 0)

            drain_gathers(0, semg[0])
            drain_idx(1, semi[1])

            plsc.subcore_barrier()
            pltpu.sync_copy(acc.at[pl.ds(sid * RPT, RPT)],
                            out.at[j, pl.ds(sid * RPT, RPT)])
        return 0

    lax.fori_loop(0, 2, one_pass, 0)


def _spmm(nslab, table_flat, rows, cols, vals, init):
    mesh = plsc.VectorSubcoreMesh(core_axis_name="c", subcore_axis_name="s",
                                  num_cores=2, num_subcores=TILES)
    fn = pl.kernel(
        functools.partial(_spmm_body, nslab),
        out_type=jax.ShapeDtypeStruct((nslab, N, 16), jnp.float32),
        mesh=mesh,
        compiler_params=pltpu.CompilerParams(needs_layout_passes=False,
                                             use_tc_tiling_on_sc=False),
        scratch_types=[
            pltpu.VMEM_SHARED((N, 16), jnp.float32),
            pltpu.VMEM((2, K, B), jnp.int32),
            pltpu.VMEM((2, K, B), jnp.int32),
            pltpu.VMEM((2, K, B), jnp.float32),
            pltpu.VMEM((2, CHUNK, 16), jnp.float32),
            pltpu.SemaphoreType.DMA,
            pltpu.SemaphoreType.DMA,
            pltpu.SemaphoreType.DMA,
            pltpu.SemaphoreType.DMA,
        ],
    )
    return fn(table_flat, rows, cols, vals, init)


def kernel(x, adj_indices, adj_values, W1, b1, W2, b2):
    rows = adj_indices[0]
    cols = adj_indices[1]
    pad = E - NNZ
    rows_p = jnp.pad(rows, (0, pad)).reshape(ER, B)
    cols_p = jnp.pad(cols, (0, pad)).reshape(ER, B)
    vals_p = jnp.pad(adj_values, (0, pad)).reshape(ER, B)

    w2p = jnp.pad(W2, ((0, 0), (0, NCPAD - NCLASS)))
    b2p = jnp.pad(b2, (0, NCPAD - NCLASS))

    xw = _mm1(x, W1)
    init0 = jnp.zeros((4, RPT, 16), jnp.float32)
    g1 = _spmm(4, xw.reshape(4 * N, 16), rows_p, cols_p, vals_p, init0)

    t = _mm2(g1, w2p, b1.reshape(1, NHID))
    init2 = jnp.broadcast_to(b2p.reshape(3, 1, 16), (3, RPT, 16))
    out3 = _spmm(3, t.reshape(3 * N, 16), rows_p, cols_p, vals_p, init2)

    return _asm(out3)
